# Initial kernel scaffold; baseline (speedup 1.0000x reference)
import jax
import jax.numpy as jnp
from jax.experimental import pallas as pl


def kernel(emb, labels, sbj):
    raise NotImplementedError("write your pallas kernel here")


# speedup vs baseline: 4.0729x; 4.0729x over previous
import jax
import jax.numpy as jnp
from jax.experimental import pallas as pl
from jax.experimental.pallas import tpu as pltpu

_MARGIN = 0.8
_BLK = 256
_BIG = 1e30


def _triplet_kernel(a_ref, e_ref, key_ref, sbj_ref, sum_ref, cnt_ref):
    i = pl.program_id(0)
    a = a_ref[...]
    e = e_ref[...]
    B = e.shape[0]
    blk = a.shape[0]

    g = jax.lax.dot_general(
        a, e, (((1,), (1,)), ((), ())), preferred_element_type=jnp.float32
    )
    sq_all = jnp.sum(e * e, axis=1)
    sq_blk = jnp.sum(a * a, axis=1)
    d2 = (sq_blk[:, None] - 2.0 * g) + sq_all[None, :]

    key = key_ref[0, :]
    sbj = sbj_ref[0, :]
    key_r = key_ref[0, pl.ds(i * blk, blk)]
    sbj_r = sbj_ref[0, pl.ds(i * blk, blk)]

    key_eq = key_r[:, None] == key[None, :]
    sbj_eq = sbj_r[:, None] == sbj[None, :]
    row = i * blk + jax.lax.broadcasted_iota(jnp.int32, (blk, B), 0)
    col = jax.lax.broadcasted_iota(jnp.int32, (blk, B), 1)
    pos = key_eq & (row != col)
    neg = sbj_eq & jnp.logical_not(key_eq)

    dpos2 = jnp.max(jnp.where(pos, d2, -1.0), axis=1)
    dneg2 = jnp.min(jnp.where(neg, d2, _BIG), axis=1)
    valid = (dpos2 >= 0.0) & (dneg2 < 1e29)

    dp = jnp.sqrt(jnp.maximum(dpos2, 0.0))
    dn = jnp.sqrt(jnp.maximum(dneg2, 0.0))
    per = jnp.maximum(dp - dn + _MARGIN, 0.0)
    psum = jnp.sum(jnp.where(valid, per, 0.0))
    pcnt = jnp.sum(valid.astype(jnp.float32))

    @pl.when(i == 0)
    def _():
        sum_ref[...] = jnp.zeros((1, 1), jnp.float32)
        cnt_ref[...] = jnp.zeros((1, 1), jnp.float32)

    sum_ref[...] += psum.reshape(1, 1)
    cnt_ref[...] += pcnt.reshape(1, 1)


def kernel(emb, labels, sbj):
    B, D = emb.shape
    lbl32 = labels.astype(jnp.int32)
    sbj32 = sbj.astype(jnp.int32)
    key2 = (sbj32 * 8 + lbl32).reshape(1, B)
    sbj2 = sbj32.reshape(1, B)
    grid = B // _BLK
    s, c = pl.pallas_call(
        _triplet_kernel,
        grid=(grid,),
        in_specs=[
            pl.BlockSpec((_BLK, D), lambda i: (i, 0)),
            pl.BlockSpec((B, D), lambda i: (0, 0)),
            pl.BlockSpec((1, B), lambda i: (0, 0)),
            pl.BlockSpec((1, B), lambda i: (0, 0)),
        ],
        out_specs=[
            pl.BlockSpec((1, 1), lambda i: (0, 0)),
            pl.BlockSpec((1, 1), lambda i: (0, 0)),
        ],
        out_shape=[
            jax.ShapeDtypeStruct((1, 1), jnp.float32),
            jax.ShapeDtypeStruct((1, 1), jnp.float32),
        ],
    )(emb, emb, key2, sbj2)
    return s[0, 0] / jnp.maximum(c[0, 0], 1.0)


# speedup vs baseline: 4.5476x; 1.1165x over previous
import jax
import jax.numpy as jnp
from jax.experimental import pallas as pl
from jax.experimental.pallas import tpu as pltpu

_MARGIN = 0.8
_BLK = 512
_BIG = 1e30


def _triplet_kernel(a_ref, e_ref, key_ref, sbj_ref, sum_ref, cnt_ref):
    i = pl.program_id(0)
    a = a_ref[...]
    e = e_ref[...]
    B = e.shape[0]
    blk = a.shape[0]

    g = jax.lax.dot_general(
        a, e, (((1,), (1,)), ((), ())), preferred_element_type=jnp.float32
    )
    sq_all = jnp.sum(e * e, axis=1)
    sq_blk = jnp.sum(a * a, axis=1)
    d2 = (sq_blk[:, None] - 2.0 * g) + sq_all[None, :]

    key = key_ref[0, :]
    sbj = sbj_ref[0, :]
    key_r = key_ref[0, pl.ds(i * blk, blk)]
    sbj_r = sbj_ref[0, pl.ds(i * blk, blk)]

    key_eq = key_r[:, None] == key[None, :]
    sbj_eq = sbj_r[:, None] == sbj[None, :]
    row = i * blk + jax.lax.broadcasted_iota(jnp.int32, (blk, B), 0)
    col = jax.lax.broadcasted_iota(jnp.int32, (blk, B), 1)
    pos = key_eq & (row != col)
    neg = sbj_eq & jnp.logical_not(key_eq)

    dpos2 = jnp.max(jnp.where(pos, d2, -1.0), axis=1)
    dneg2 = jnp.min(jnp.where(neg, d2, _BIG), axis=1)
    valid = (dpos2 >= 0.0) & (dneg2 < 1e29)

    dp = jnp.sqrt(jnp.maximum(dpos2, 0.0))
    dn = jnp.sqrt(jnp.maximum(dneg2, 0.0))
    per = jnp.maximum(dp - dn + _MARGIN, 0.0)
    psum = jnp.sum(jnp.where(valid, per, 0.0))
    pcnt = jnp.sum(valid.astype(jnp.float32))

    @pl.when(i == 0)
    def _():
        sum_ref[...] = jnp.zeros((1, 1), jnp.float32)
        cnt_ref[...] = jnp.zeros((1, 1), jnp.float32)

    sum_ref[...] += psum.reshape(1, 1)
    cnt_ref[...] += pcnt.reshape(1, 1)


def kernel(emb, labels, sbj):
    B, D = emb.shape
    lbl32 = labels.astype(jnp.int32)
    sbj32 = sbj.astype(jnp.int32)
    key2 = (sbj32 * 8 + lbl32).reshape(1, B)
    sbj2 = sbj32.reshape(1, B)
    grid = B // _BLK
    s, c = pl.pallas_call(
        _triplet_kernel,
        grid=(grid,),
        in_specs=[
            pl.BlockSpec((_BLK, D), lambda i: (i, 0)),
            pl.BlockSpec((B, D), lambda i: (0, 0)),
            pl.BlockSpec((1, B), lambda i: (0, 0)),
            pl.BlockSpec((1, B), lambda i: (0, 0)),
        ],
        out_specs=[
            pl.BlockSpec((1, 1), lambda i: (0, 0)),
            pl.BlockSpec((1, 1), lambda i: (0, 0)),
        ],
        out_shape=[
            jax.ShapeDtypeStruct((1, 1), jnp.float32),
            jax.ShapeDtypeStruct((1, 1), jnp.float32),
        ],
    )(emb, emb, key2, sbj2)
    return s[0, 0] / jnp.maximum(c[0, 0], 1.0)
